# baseline (device time: 216872 ns/iter reference)
import jax
import jax.numpy as jnp
from jax import lax
from jax.experimental import pallas as pl
from jax.experimental.pallas import tpu as pltpu

N_DEV = 32
M = 2048
K = 1024
N = 1024
CHUNK = M // N_DEV


def kernel(t, W):
    def body(t_ref, w_ref, out_ref,
             tb_ref, wb_ref,
             rs_comm, ag_comm, send_buf,
             send_sems, rs_recv_sems, ag_recv_sems):
        my = lax.axis_index("i")
        left = lax.rem(my - 1 + N_DEV, N_DEV)
        right = lax.rem(my + 1, N_DEV)

        barrier_sem = pltpu.get_barrier_semaphore()
        for nbr in (left, right):
            pl.semaphore_signal(
                barrier_sem, inc=1,
                device_id=(nbr,), device_id_type=pl.DeviceIdType.MESH,
            )
        pl.semaphore_wait(barrier_sem, 2)

        tb_ref[...] = t_ref[...].astype(jnp.bfloat16)
        wb_ref[...] = w_ref[...].astype(jnp.bfloat16)

        def rows(c):
            return pl.ds(c * CHUNK, CHUNK)

        for s in range(N_DEV - 1):
            slot = s % 2
            c = lax.rem(my - s + N_DEV, N_DEV)
            if s == 0:
                send_buf[slot, :, :] = tb_ref[rows(c), :]
            else:
                send_buf[slot, :, :] = rs_comm[s - 1, :, :] + tb_ref[rows(c), :]
            rdma = pltpu.make_async_remote_copy(
                src_ref=send_buf.at[slot],
                dst_ref=rs_comm.at[s],
                send_sem=send_sems.at[slot],
                recv_sem=rs_recv_sems.at[s],
                device_id=(right,),
                device_id_type=pl.DeviceIdType.MESH,
            )
            rdma.start()
            rdma.wait()

        my_chunk = lax.rem(my + 1, N_DEV)
        reduced = rs_comm[N_DEV - 2, :, :] + tb_ref[rows(my_chunk), :]

        y = jnp.dot(reduced, wb_ref[...], preferred_element_type=jnp.float32)
        out_ref[rows(my_chunk), :] = y

        send_buf[0, :, :] = y.astype(jnp.bfloat16)
        for h in range(N_DEV - 1):
            src = send_buf.at[0] if h == 0 else ag_comm.at[h - 1]
            rdma = pltpu.make_async_remote_copy(
                src_ref=src,
                dst_ref=ag_comm.at[h],
                send_sem=send_sems.at[h % 2],
                recv_sem=ag_recv_sems.at[h],
                device_id=(right,),
                device_id_type=pl.DeviceIdType.MESH,
            )
            rdma.start()
            rdma.wait()
            org = lax.rem(my - h + N_DEV, N_DEV)
            out_ref[rows(org), :] = ag_comm[h, :, :].astype(jnp.float32)

    return pl.pallas_call(
        body,
        out_shape=jax.ShapeDtypeStruct((M, N), jnp.float32),
        in_specs=[
            pl.BlockSpec(memory_space=pltpu.VMEM),
            pl.BlockSpec(memory_space=pltpu.VMEM),
        ],
        out_specs=pl.BlockSpec(memory_space=pltpu.VMEM),
        scratch_shapes=[
            pltpu.VMEM((M, K), jnp.bfloat16),
            pltpu.VMEM((K, N), jnp.bfloat16),
            pltpu.VMEM((N_DEV - 1, CHUNK, K), jnp.bfloat16),
            pltpu.VMEM((N_DEV - 1, CHUNK, N), jnp.bfloat16),
            pltpu.VMEM((2, CHUNK, K), jnp.bfloat16),
            pltpu.SemaphoreType.DMA((2,)),
            pltpu.SemaphoreType.DMA((N_DEV - 1,)),
            pltpu.SemaphoreType.DMA((N_DEV - 1,)),
        ],
        compiler_params=pltpu.CompilerParams(collective_id=0),
    )(t, W)


# device time: 132087 ns/iter; 1.6419x vs baseline; 1.6419x over previous
import jax
import jax.numpy as jnp
from jax import lax
from jax.experimental import pallas as pl
from jax.experimental.pallas import tpu as pltpu

N_DEV = 32
M = 2048
K = 1024
N = 1024


def kernel(t, W):
    def body(t_ref, w_ref, out_ref,
             acc_ref, wb_ref,
             rs_bufs, ag_bufs,
             send_sems, recv_sems):
        my = lax.axis_index("i")
        z = my // 8
        r = my - z * 8
        y = r // 2
        q = r - y * 2
        x = jnp.where(y % 2 == 0, q, 1 - q)

        def lid(xx, yy, zz):
            qq = jnp.where(yy % 2 == 0, xx, 1 - xx)
            return zz * 8 + yy * 2 + qq

        phases = [
            (lid(1 - x, y, z), x),
            (lid(x, y ^ 1, z), y % 2),
            (lid(x, y ^ 2, z), y // 2),
            (lid(x, y, z ^ 1), z % 2),
            (lid(x, y, z ^ 2), z // 2),
        ]

        barrier_sem = pltpu.get_barrier_semaphore()
        for p, _ in phases:
            pl.semaphore_signal(
                barrier_sem, inc=1,
                device_id=(p,), device_id_type=pl.DeviceIdType.MESH,
            )
        pl.semaphore_wait(barrier_sem, len(phases))

        acc_ref[...] = t_ref[...].astype(jnp.bfloat16)
        wb_ref[...] = w_ref[...].astype(jnp.bfloat16)

        base = jnp.int32(0)
        size = M
        for s, (partner, bit) in enumerate(phases):
            half = size // 2
            send_base = base + half * (1 - bit)
            keep_base = base + half * bit
            rdma = pltpu.make_async_remote_copy(
                src_ref=acc_ref.at[pl.ds(send_base, half), :],
                dst_ref=rs_bufs[s],
                send_sem=send_sems.at[s],
                recv_sem=recv_sems.at[s],
                device_id=(partner,),
                device_id_type=pl.DeviceIdType.MESH,
            )
            rdma.start()
            rdma.wait()
            acc_ref[pl.ds(keep_base, half), :] = (
                acc_ref[pl.ds(keep_base, half), :] + rs_bufs[s][...]
            )
            base = keep_base
            size = half

        yv = jnp.dot(
            acc_ref[pl.ds(base, size), :], wb_ref[...],
            preferred_element_type=jnp.float32,
        )
        out_ref[pl.ds(base, size), :] = yv
        acc_ref[pl.ds(base, size), :] = yv.astype(jnp.bfloat16)

        for s, (partner, bit) in reversed(list(enumerate(phases))):
            sib_base = base + size * (1 - 2 * bit)
            rdma = pltpu.make_async_remote_copy(
                src_ref=acc_ref.at[pl.ds(base, size), :],
                dst_ref=ag_bufs[s],
                send_sem=send_sems.at[5 + s],
                recv_sem=recv_sems.at[5 + s],
                device_id=(partner,),
                device_id_type=pl.DeviceIdType.MESH,
            )
            rdma.start()
            rdma.wait()
            acc_ref[pl.ds(sib_base, size), :] = ag_bufs[s][...]
            out_ref[pl.ds(sib_base, size), :] = ag_bufs[s][...].astype(
                jnp.float32
            )
            base = base - size * bit
            size = size * 2

    rs_shapes = [pltpu.VMEM((M >> (s + 1), K), jnp.bfloat16) for s in range(5)]
    ag_shapes = [pltpu.VMEM((M >> (s + 1), N), jnp.bfloat16) for s in range(5)]
    return pl.pallas_call(
        body,
        out_shape=jax.ShapeDtypeStruct((M, N), jnp.float32),
        in_specs=[
            pl.BlockSpec(memory_space=pltpu.VMEM),
            pl.BlockSpec(memory_space=pltpu.VMEM),
        ],
        out_specs=pl.BlockSpec(memory_space=pltpu.VMEM),
        scratch_shapes=[
            pltpu.VMEM((M, K), jnp.bfloat16),
            pltpu.VMEM((K, N), jnp.bfloat16),
            rs_shapes,
            ag_shapes,
            pltpu.SemaphoreType.DMA((10,)),
            pltpu.SemaphoreType.DMA((10,)),
        ],
        compiler_params=pltpu.CompilerParams(collective_id=0),
    )(t, W)


# device time: 107026 ns/iter; 2.0263x vs baseline; 1.2342x over previous
import jax
import jax.numpy as jnp
from jax import lax
from jax.experimental import pallas as pl
from jax.experimental.pallas import tpu as pltpu

M = 2048
K = 1024
N = 1024
HK = K // 2
CH = 64
STEPS = 5


def kernel(t, W):
    def body(t_ref, w_ref, out_ref,
             accA, accB, wb_ref,
             rs_bufs, ag_bufs,
             shufin, shufout, shufin2,
             rs_send_sems, rs_recv_sems,
             ag_send_sems, ag_recv_sems,
             sh_send_sems, sh_recv_sems):
        my = lax.axis_index("i")
        z = my // 8
        r = my - z * 8
        y = r // 2
        q = r - y * 2
        x = jnp.where(y % 2 == 0, q, 1 - q)

        def lid(xx, yy, zz):
            qq = jnp.where(yy % 2 == 0, xx, 1 - xx)
            return zz * 8 + yy * 2 + qq

        PX = (lid(1 - x, y, z), x)
        PY1 = (lid(x, y ^ 1, z), y % 2)
        PY2 = (lid(x, y ^ 2, z), y // 2)
        PZ1 = (lid(x, y, z ^ 1), z % 2)
        PZ2 = (lid(x, y, z ^ 2), z // 2)
        ordA = [PX, PY1, PY2, PZ1, PZ2]
        ordB = [PZ1, PZ2, PX, PY1, PY2]

        Dp = lid(y // 2, z, (y % 2) * 2 + x)
        Dst = lid(z % 2, x * 2 + z // 2, y)
        pred = Dp != my

        barrier_sem = pltpu.get_barrier_semaphore()
        for p, _ in ordA:
            pl.semaphore_signal(
                barrier_sem, inc=1,
                device_id=(p,), device_id_type=pl.DeviceIdType.MESH,
            )

        @pl.when(pred)
        def _():
            for p in (Dp, Dst):
                pl.semaphore_signal(
                    barrier_sem, inc=1,
                    device_id=(p,), device_id_type=pl.DeviceIdType.MESH,
                )

        @pl.when(jnp.logical_not(pred))
        def _():
            pl.semaphore_signal(barrier_sem, inc=2)

        pl.semaphore_wait(barrier_sem, 7)

        accA[...] = t_ref[:, 0:HK].astype(jnp.bfloat16)
        accB[...] = t_ref[:, HK:K].astype(jnp.bfloat16)
        wb_ref[...] = w_ref[...].astype(jnp.bfloat16)

        baseA = jnp.int32(0)
        baseB = jnp.int32(0)
        size = M
        for s in range(STEPS):
            half = size // 2
            pA, bA = ordA[s]
            pB, bB = ordB[s]
            rdmaA = pltpu.make_async_remote_copy(
                src_ref=accA.at[pl.ds(baseA + half * (1 - bA), half), :],
                dst_ref=rs_bufs[s],
                send_sem=rs_send_sems.at[s],
                recv_sem=rs_recv_sems.at[s],
                device_id=(pA,),
                device_id_type=pl.DeviceIdType.MESH,
            )
            rdmaB = pltpu.make_async_remote_copy(
                src_ref=accB.at[pl.ds(baseB + half * (1 - bB), half), :],
                dst_ref=rs_bufs[STEPS + s],
                send_sem=rs_send_sems.at[STEPS + s],
                recv_sem=rs_recv_sems.at[STEPS + s],
                device_id=(pB,),
                device_id_type=pl.DeviceIdType.MESH,
            )
            rdmaA.start()
            rdmaB.start()
            rdmaA.wait()
            rdmaB.wait()
            keepA = baseA + half * bA
            keepB = baseB + half * bB
            accA[pl.ds(keepA, half), :] = (
                accA[pl.ds(keepA, half), :] + rs_bufs[s][...]
            )
            accB[pl.ds(keepB, half), :] = (
                accB[pl.ds(keepB, half), :] + rs_bufs[STEPS + s][...]
            )
            baseA = keepA
            baseB = keepB
            size = half

        @pl.when(pred)
        def _():
            rdma = pltpu.make_async_remote_copy(
                src_ref=accB.at[pl.ds(baseB, CH), :],
                dst_ref=shufin,
                send_sem=sh_send_sems.at[0],
                recv_sem=sh_recv_sems.at[0],
                device_id=(Dst,),
                device_id_type=pl.DeviceIdType.MESH,
            )
            rdma.start()
            rdma.wait()

        @pl.when(jnp.logical_not(pred))
        def _():
            shufin[...] = accB[pl.ds(baseB, CH), :]

        yv = jnp.dot(
            accA[pl.ds(baseA, CH), :], wb_ref[0:HK, :],
            preferred_element_type=jnp.float32,
        ) + jnp.dot(
            shufin[...], wb_ref[HK:K, :],
            preferred_element_type=jnp.float32,
        )
        out_ref[pl.ds(baseA, CH), :] = yv
        accA[pl.ds(baseA, CH), :] = yv[:, 0:HK].astype(jnp.bfloat16)
        shufout[...] = yv[:, HK:N].astype(jnp.bfloat16)

        @pl.when(pred)
        def _():
            rdma = pltpu.make_async_remote_copy(
                src_ref=shufout,
                dst_ref=shufin2,
                send_sem=sh_send_sems.at[1],
                recv_sem=sh_recv_sems.at[1],
                device_id=(Dp,),
                device_id_type=pl.DeviceIdType.MESH,
            )
            rdma.start()
            rdma.wait()

        @pl.when(jnp.logical_not(pred))
        def _():
            shufin2[...] = shufout[...]

        accB[pl.ds(baseB, CH), :] = shufin2[...]
        out_ref[pl.ds(baseB, CH), pl.ds(HK, HK)] = shufin2[...].astype(
            jnp.float32
        )

        for j in range(STEPS):
            sz = CH << j
            pA, bA = ordA[STEPS - 1 - j]
            pB, bB = ordB[STEPS - 1 - j]
            rdmaA = pltpu.make_async_remote_copy(
                src_ref=accA.at[pl.ds(baseA, sz), :],
                dst_ref=ag_bufs[j],
                send_sem=ag_send_sems.at[j],
                recv_sem=ag_recv_sems.at[j],
                device_id=(pA,),
                device_id_type=pl.DeviceIdType.MESH,
            )
            rdmaB = pltpu.make_async_remote_copy(
                src_ref=accB.at[pl.ds(baseB, sz), :],
                dst_ref=ag_bufs[STEPS + j],
                send_sem=ag_send_sems.at[STEPS + j],
                recv_sem=ag_recv_sems.at[STEPS + j],
                device_id=(pB,),
                device_id_type=pl.DeviceIdType.MESH,
            )
            rdmaA.start()
            rdmaB.start()
            rdmaA.wait()
            rdmaB.wait()
            sibA = baseA + sz * (1 - 2 * bA)
            sibB = baseB + sz * (1 - 2 * bB)
            accA[pl.ds(sibA, sz), :] = ag_bufs[j][...]
            out_ref[pl.ds(sibA, sz), pl.ds(0, HK)] = ag_bufs[j][...].astype(
                jnp.float32
            )
            accB[pl.ds(sibB, sz), :] = ag_bufs[STEPS + j][...]
            out_ref[pl.ds(sibB, sz), pl.ds(HK, HK)] = (
                ag_bufs[STEPS + j][...].astype(jnp.float32)
            )
            baseA = baseA - sz * bA
            baseB = baseB - sz * bB

    rs_shapes = [
        pltpu.VMEM((M >> (s + 1), HK), jnp.bfloat16) for s in range(STEPS)
    ] * 2
    ag_shapes = [
        pltpu.VMEM((CH << j, HK), jnp.bfloat16) for j in range(STEPS)
    ] * 2
    return pl.pallas_call(
        body,
        out_shape=jax.ShapeDtypeStruct((M, N), jnp.float32),
        in_specs=[
            pl.BlockSpec(memory_space=pltpu.VMEM),
            pl.BlockSpec(memory_space=pltpu.VMEM),
        ],
        out_specs=pl.BlockSpec(memory_space=pltpu.VMEM),
        scratch_shapes=[
            pltpu.VMEM((M, HK), jnp.bfloat16),
            pltpu.VMEM((M, HK), jnp.bfloat16),
            pltpu.VMEM((K, N), jnp.bfloat16),
            rs_shapes,
            ag_shapes,
            pltpu.VMEM((CH, HK), jnp.bfloat16),
            pltpu.VMEM((CH, HK), jnp.bfloat16),
            pltpu.VMEM((CH, HK), jnp.bfloat16),
            pltpu.SemaphoreType.DMA((2 * STEPS,)),
            pltpu.SemaphoreType.DMA((2 * STEPS,)),
            pltpu.SemaphoreType.DMA((2 * STEPS,)),
            pltpu.SemaphoreType.DMA((2 * STEPS,)),
            pltpu.SemaphoreType.DMA((2,)),
            pltpu.SemaphoreType.DMA((2,)),
        ],
        compiler_params=pltpu.CompilerParams(collective_id=0),
    )(t, W)


# device time: 98108 ns/iter; 2.2105x vs baseline; 1.0909x over previous
import jax
import jax.numpy as jnp
from jax import lax
from jax.experimental import pallas as pl
from jax.experimental.pallas import tpu as pltpu

M = 2048
K = 1024
N = 1024
HK = K // 2
CH = 64
STEPS = 5


def kernel(t, W):
    def body(t_ref, w_ref, out_ref,
             accA, accB, wb_ref,
             rs_bufs, ag_bufs,
             shufin, shufout, shufin2,
             rs_send_sems, rs_recv_sems,
             ag_send_sems, ag_recv_sems,
             sh_send_sems, sh_recv_sems):
        my = lax.axis_index("i")
        z = my // 8
        r = my - z * 8
        y = r // 2
        q = r - y * 2
        x = jnp.where(y % 2 == 0, q, 1 - q)

        def lid(xx, yy, zz):
            qq = jnp.where(yy % 2 == 0, xx, 1 - xx)
            return zz * 8 + yy * 2 + qq

        PX = (lid(1 - x, y, z), x)
        PY1 = (lid(x, y ^ 1, z), y % 2)
        PY2 = (lid(x, y ^ 2, z), y // 2)
        PZ1 = (lid(x, y, z ^ 1), z % 2)
        PZ2 = (lid(x, y, z ^ 2), z // 2)
        ordA = [PX, PY1, PY2, PZ1, PZ2]
        ordB = [PZ1, PZ2, PX, PY1, PY2]

        Dp = lid(y // 2, z, (y % 2) * 2 + x)
        Dst = lid(z % 2, x * 2 + z // 2, y)
        pred = Dp != my

        barrier_sem = pltpu.get_barrier_semaphore()
        for p, _ in ordA:
            pl.semaphore_signal(
                barrier_sem, inc=1,
                device_id=(p,), device_id_type=pl.DeviceIdType.MESH,
            )

        @pl.when(pred)
        def _():
            for p in (Dp, Dst):
                pl.semaphore_signal(
                    barrier_sem, inc=1,
                    device_id=(p,), device_id_type=pl.DeviceIdType.MESH,
                )

        @pl.when(jnp.logical_not(pred))
        def _():
            pl.semaphore_signal(barrier_sem, inc=2)

        pl.semaphore_wait(barrier_sem, 7)

        def mk(src, dst, ssem, rsem, dev):
            return pltpu.make_async_remote_copy(
                src_ref=src, dst_ref=dst, send_sem=ssem, recv_sem=rsem,
                device_id=(dev,), device_id_type=pl.DeviceIdType.MESH,
            )

        h0 = M // 2
        pA0, bA0 = ordA[0]
        pB0, bB0 = ordB[0]
        sendA = h0 * (1 - bA0)
        sendB = h0 * (1 - bB0)
        accA[pl.ds(sendA, h0), :] = t_ref[pl.ds(sendA, h0), 0:HK].astype(
            jnp.bfloat16
        )
        rdmaA = mk(accA.at[pl.ds(sendA, h0), :], rs_bufs[0],
                   rs_send_sems.at[0], rs_recv_sems.at[0], pA0)
        rdmaA.start()
        accB[pl.ds(sendB, h0), :] = t_ref[pl.ds(sendB, h0), HK:K].astype(
            jnp.bfloat16
        )
        rdmaB = mk(accB.at[pl.ds(sendB, h0), :], rs_bufs[STEPS],
                   rs_send_sems.at[STEPS], rs_recv_sems.at[STEPS], pB0)
        rdmaB.start()
        baseA = jnp.int32(h0) * bA0
        baseB = jnp.int32(h0) * bB0
        accA[pl.ds(baseA, h0), :] = t_ref[pl.ds(baseA, h0), 0:HK].astype(
            jnp.bfloat16
        )
        accB[pl.ds(baseB, h0), :] = t_ref[pl.ds(baseB, h0), HK:K].astype(
            jnp.bfloat16
        )
        wb_ref[...] = w_ref[...].astype(jnp.bfloat16)

        for s in range(STEPS):
            half = M >> (s + 1)
            nxt = half // 2
            rdmaA.wait()
            accA[pl.ds(baseA, half), :] = (
                accA[pl.ds(baseA, half), :] + rs_bufs[s][...]
            )
            if s + 1 < STEPS:
                pA, bA = ordA[s + 1]
                rdmaA = mk(
                    accA.at[pl.ds(baseA + nxt * (1 - bA), nxt), :],
                    rs_bufs[s + 1],
                    rs_send_sems.at[s + 1], rs_recv_sems.at[s + 1], pA,
                )
                rdmaA.start()
                baseA = baseA + nxt * bA
            rdmaB.wait()
            accB[pl.ds(baseB, half), :] = (
                accB[pl.ds(baseB, half), :] + rs_bufs[STEPS + s][...]
            )
            if s + 1 < STEPS:
                pB, bB = ordB[s + 1]
                rdmaB = mk(
                    accB.at[pl.ds(baseB + nxt * (1 - bB), nxt), :],
                    rs_bufs[STEPS + s + 1],
                    rs_send_sems.at[STEPS + s + 1],
                    rs_recv_sems.at[STEPS + s + 1], pB,
                )
                rdmaB.start()
                baseB = baseB + nxt * bB

        rdma_pre = mk(accB.at[pl.ds(baseB, CH), :], shufin,
                      sh_send_sems.at[0], sh_recv_sems.at[0], Dst)

        @pl.when(pred)
        def _():
            rdma_pre.start()

        @pl.when(jnp.logical_not(pred))
        def _():
            shufin[...] = accB[pl.ds(baseB, CH), :]

        yvA = jnp.dot(
            accA[pl.ds(baseA, CH), :], wb_ref[0:HK, :],
            preferred_element_type=jnp.float32,
        )

        @pl.when(pred)
        def _():
            rdma_pre.wait()

        yv = yvA + jnp.dot(
            shufin[...], wb_ref[HK:K, :], preferred_element_type=jnp.float32,
        )
        accA[pl.ds(baseA, CH), :] = yv[:, 0:HK].astype(jnp.bfloat16)
        shufout[...] = yv[:, HK:N].astype(jnp.bfloat16)

        rdma_post = mk(shufout, shufin2,
                       sh_send_sems.at[1], sh_recv_sems.at[1], Dp)

        @pl.when(pred)
        def _():
            rdma_post.start()

        @pl.when(jnp.logical_not(pred))
        def _():
            shufin2[...] = shufout[...]

        pA, bA = ordA[STEPS - 1]
        rdmaA = mk(accA.at[pl.ds(baseA, CH), :], ag_bufs[0],
                   ag_send_sems.at[0], ag_recv_sems.at[0], pA)
        rdmaA.start()
        out_ref[pl.ds(baseA, CH), :] = yv

        @pl.when(pred)
        def _():
            rdma_post.wait()

        accB[pl.ds(baseB, CH), :] = shufin2[...]
        pB, bB = ordB[STEPS - 1]
        rdmaB = mk(accB.at[pl.ds(baseB, CH), :], ag_bufs[STEPS],
                   ag_send_sems.at[STEPS], ag_recv_sems.at[STEPS], pB)
        rdmaB.start()
        out_ref[pl.ds(baseB, CH), pl.ds(HK, HK)] = shufin2[...].astype(
            jnp.float32
        )

        sibA = jnp.int32(0)
        sibB = jnp.int32(0)
        for j in range(STEPS):
            sz = CH << j
            _, bA = ordA[STEPS - 1 - j]
            _, bB = ordB[STEPS - 1 - j]
            sibA = baseA + sz * (1 - 2 * bA)
            sibB = baseB + sz * (1 - 2 * bB)
            newA = baseA - sz * bA
            newB = baseB - sz * bB
            rdmaA.wait()
            if j + 1 < STEPS:
                accA[pl.ds(sibA, sz), :] = ag_bufs[j][...]
                pA, _ = ordA[STEPS - 2 - j]
                rdmaA = mk(accA.at[pl.ds(newA, 2 * sz), :], ag_bufs[j + 1],
                           ag_send_sems.at[j + 1], ag_recv_sems.at[j + 1], pA)
                rdmaA.start()
            out_ref[pl.ds(sibA, sz), pl.ds(0, HK)] = ag_bufs[j][...].astype(
                jnp.float32
            )
            rdmaB.wait()
            if j + 1 < STEPS:
                accB[pl.ds(sibB, sz), :] = ag_bufs[STEPS + j][...]
                pB, _ = ordB[STEPS - 2 - j]
                rdmaB = mk(accB.at[pl.ds(newB, 2 * sz), :],
                           ag_bufs[STEPS + j + 1],
                           ag_send_sems.at[STEPS + j + 1],
                           ag_recv_sems.at[STEPS + j + 1], pB)
                rdmaB.start()
            out_ref[pl.ds(sibB, sz), pl.ds(HK, HK)] = (
                ag_bufs[STEPS + j][...].astype(jnp.float32)
            )
            baseA = newA
            baseB = newB

    rs_shapes = [
        pltpu.VMEM((M >> (s + 1), HK), jnp.bfloat16) for s in range(STEPS)
    ] * 2
    ag_shapes = [
        pltpu.VMEM((CH << j, HK), jnp.bfloat16) for j in range(STEPS)
    ] * 2
    return pl.pallas_call(
        body,
        out_shape=jax.ShapeDtypeStruct((M, N), jnp.float32),
        in_specs=[
            pl.BlockSpec(memory_space=pltpu.VMEM),
            pl.BlockSpec(memory_space=pltpu.VMEM),
        ],
        out_specs=pl.BlockSpec(memory_space=pltpu.VMEM),
        scratch_shapes=[
            pltpu.VMEM((M, HK), jnp.bfloat16),
            pltpu.VMEM((M, HK), jnp.bfloat16),
            pltpu.VMEM((K, N), jnp.bfloat16),
            rs_shapes,
            ag_shapes,
            pltpu.VMEM((CH, HK), jnp.bfloat16),
            pltpu.VMEM((CH, HK), jnp.bfloat16),
            pltpu.VMEM((CH, HK), jnp.bfloat16),
            pltpu.SemaphoreType.DMA((2 * STEPS,)),
            pltpu.SemaphoreType.DMA((2 * STEPS,)),
            pltpu.SemaphoreType.DMA((2 * STEPS,)),
            pltpu.SemaphoreType.DMA((2 * STEPS,)),
            pltpu.SemaphoreType.DMA((2,)),
            pltpu.SemaphoreType.DMA((2,)),
        ],
        compiler_params=pltpu.CompilerParams(collective_id=0),
    )(t, W)


# device time: 81325 ns/iter; 2.6667x vs baseline; 1.2064x over previous
import jax
import jax.numpy as jnp
from jax import lax
from jax.experimental import pallas as pl
from jax.experimental.pallas import tpu as pltpu

M = 2048
K = 1024
N = 1024
CH = 64
STEPS = 5
WIDTHS = (384, 256, 384)
COLS = (0, 384, 640)


def kernel(t, W):
    def body(t_ref, w_ref, out_ref,
             accs, wb_ref,
             rs_bufs, ag_bufs,
             shufin_b, shufin_c, shufout_b, shufout_c,
             shufin2_b, shufin2_c,
             rs_send_sems, rs_recv_sems,
             ag_send_sems, ag_recv_sems,
             sh_send_sems, sh_recv_sems):
        my = lax.axis_index("i")
        z = my // 8
        r = my - z * 8
        y = r // 2
        q = r - y * 2
        x = jnp.where(y % 2 == 0, q, 1 - q)

        def lid(xx, yy, zz):
            qq = jnp.where(yy % 2 == 0, xx, 1 - xx)
            return zz * 8 + yy * 2 + qq

        PX = (lid(1 - x, y, z), x)
        PY1 = (lid(x, y ^ 1, z), y % 2)
        PY2 = (lid(x, y ^ 2, z), y // 2)
        PZ1 = (lid(x, y, z ^ 1), z % 2)
        PZ2 = (lid(x, y, z ^ 2), z // 2)
        orders = [
            [PX, PY1, PY2, PZ1, PZ2],
            [PZ1, PZ2, PX, PY1, PY2],
            [PY1, PX, PZ1, PY2, PZ2],
        ]

        Dp_b = lid(y // 2, z, (y % 2) * 2 + x)
        Dst_b = lid(z % 2, x * 2 + z // 2, y)
        Dc = lid(y % 2, (z % 2) * 2 + x, (z // 2) * 2 + y // 2)
        pred_b = Dp_b != my
        pred_c = Dc != my

        barrier_sem = pltpu.get_barrier_semaphore()
        for p, _ in orders[0]:
            pl.semaphore_signal(
                barrier_sem, inc=1,
                device_id=(p,), device_id_type=pl.DeviceIdType.MESH,
            )

        @pl.when(pred_b)
        def _():
            for p in (Dp_b, Dst_b):
                pl.semaphore_signal(
                    barrier_sem, inc=1,
                    device_id=(p,), device_id_type=pl.DeviceIdType.MESH,
                )

        @pl.when(jnp.logical_not(pred_b))
        def _():
            pl.semaphore_signal(barrier_sem, inc=2)

        @pl.when(pred_c)
        def _():
            pl.semaphore_signal(
                barrier_sem, inc=1,
                device_id=(Dc,), device_id_type=pl.DeviceIdType.MESH,
            )

        @pl.when(jnp.logical_not(pred_c))
        def _():
            pl.semaphore_signal(barrier_sem, inc=1)

        pl.semaphore_wait(barrier_sem, 8)

        def mk(src, dst, ssem, rsem, dev):
            return pltpu.make_async_remote_copy(
                src_ref=src, dst_ref=dst, send_sem=ssem, recv_sem=rsem,
                device_id=(dev,), device_id_type=pl.DeviceIdType.MESH,
            )

        h0 = M // 2
        rdma = [None, None, None]
        base = [None, None, None]
        for i in range(3):
            p0, b0 = orders[i][0]
            send0 = h0 * (1 - b0)
            c0, w = COLS[i], WIDTHS[i]
            accs[i][pl.ds(send0, h0), :] = t_ref[
                pl.ds(send0, h0), c0:c0 + w
            ].astype(jnp.bfloat16)
            rdma[i] = mk(accs[i].at[pl.ds(send0, h0), :], rs_bufs[i * STEPS],
                         rs_send_sems.at[i * STEPS], rs_recv_sems.at[i * STEPS],
                         p0)
            rdma[i].start()
            base[i] = jnp.int32(h0) * b0
        for i in range(3):
            c0, w = COLS[i], WIDTHS[i]
            accs[i][pl.ds(base[i], h0), :] = t_ref[
                pl.ds(base[i], h0), c0:c0 + w
            ].astype(jnp.bfloat16)
        wb_ref[...] = w_ref[...].astype(jnp.bfloat16)

        for s in range(STEPS):
            half = M >> (s + 1)
            nxt = half // 2
            for i in range(3):
                rdma[i].wait()
                accs[i][pl.ds(base[i], half), :] = (
                    accs[i][pl.ds(base[i], half), :]
                    + rs_bufs[i * STEPS + s][...]
                )
                if s + 1 < STEPS:
                    p, b = orders[i][s + 1]
                    rdma[i] = mk(
                        accs[i].at[pl.ds(base[i] + nxt * (1 - b), nxt), :],
                        rs_bufs[i * STEPS + s + 1],
                        rs_send_sems.at[i * STEPS + s + 1],
                        rs_recv_sems.at[i * STEPS + s + 1], p,
                    )
                    rdma[i].start()
                    base[i] = base[i] + nxt * b

        baseA, baseB, baseC = base

        rdma_pre_b = mk(accs[1].at[pl.ds(baseB, CH), :], shufin_b,
                        sh_send_sems.at[0], sh_recv_sems.at[0], Dst_b)
        rdma_pre_c = mk(accs[2].at[pl.ds(baseC, CH), :], shufin_c,
                        sh_send_sems.at[1], sh_recv_sems.at[1], Dc)

        @pl.when(pred_b)
        def _():
            rdma_pre_b.start()

        @pl.when(jnp.logical_not(pred_b))
        def _():
            shufin_b[...] = accs[1][pl.ds(baseB, CH), :]

        @pl.when(pred_c)
        def _():
            rdma_pre_c.start()

        @pl.when(jnp.logical_not(pred_c))
        def _():
            shufin_c[...] = accs[2][pl.ds(baseC, CH), :]

        yv = jnp.dot(
            accs[0][pl.ds(baseA, CH), :], wb_ref[0:WIDTHS[0], :],
            preferred_element_type=jnp.float32,
        )

        @pl.when(pred_b)
        def _():
            rdma_pre_b.wait()

        yv = yv + jnp.dot(
            shufin_b[...], wb_ref[COLS[1]:COLS[1] + WIDTHS[1], :],
            preferred_element_type=jnp.float32,
        )

        @pl.when(pred_c)
        def _():
            rdma_pre_c.wait()

        yv = yv + jnp.dot(
            shufin_c[...], wb_ref[COLS[2]:K, :],
            preferred_element_type=jnp.float32,
        )
        accs[0][pl.ds(baseA, CH), :] = yv[:, 0:WIDTHS[0]].astype(jnp.bfloat16)
        shufout_b[...] = yv[:, COLS[1]:COLS[1] + WIDTHS[1]].astype(
            jnp.bfloat16
        )
        shufout_c[...] = yv[:, COLS[2]:N].astype(jnp.bfloat16)

        rdma_post_b = mk(shufout_b, shufin2_b,
                         sh_send_sems.at[2], sh_recv_sems.at[2], Dp_b)
        rdma_post_c = mk(shufout_c, shufin2_c,
                         sh_send_sems.at[3], sh_recv_sems.at[3], Dc)

        @pl.when(pred_b)
        def _():
            rdma_post_b.start()

        @pl.when(jnp.logical_not(pred_b))
        def _():
            shufin2_b[...] = shufout_b[...]

        @pl.when(pred_c)
        def _():
            rdma_post_c.start()

        @pl.when(jnp.logical_not(pred_c))
        def _():
            shufin2_c[...] = shufout_c[...]

        pA, _ = orders[0][STEPS - 1]
        rdma[0] = mk(accs[0].at[pl.ds(baseA, CH), :], ag_bufs[0],
                     ag_send_sems.at[0], ag_recv_sems.at[0], pA)
        rdma[0].start()
        out_ref[pl.ds(baseA, CH), :] = yv

        @pl.when(pred_b)
        def _():
            rdma_post_b.wait()

        accs[1][pl.ds(baseB, CH), :] = shufin2_b[...]
        pB, _ = orders[1][STEPS - 1]
        rdma[1] = mk(accs[1].at[pl.ds(baseB, CH), :], ag_bufs[STEPS],
                     ag_send_sems.at[STEPS], ag_recv_sems.at[STEPS], pB)
        rdma[1].start()
        out_ref[pl.ds(baseB, CH), pl.ds(COLS[1], WIDTHS[1])] = (
            shufin2_b[...].astype(jnp.float32)
        )

        @pl.when(pred_c)
        def _():
            rdma_post_c.wait()

        accs[2][pl.ds(baseC, CH), :] = shufin2_c[...]
        pC, _ = orders[2][STEPS - 1]
        rdma[2] = mk(accs[2].at[pl.ds(baseC, CH), :], ag_bufs[2 * STEPS],
                     ag_send_sems.at[2 * STEPS], ag_recv_sems.at[2 * STEPS],
                     pC)
        rdma[2].start()
        out_ref[pl.ds(baseC, CH), pl.ds(COLS[2], WIDTHS[2])] = (
            shufin2_c[...].astype(jnp.float32)
        )

        base = [baseA, baseB, baseC]

        for j in range(STEPS):
            sz = CH << j
            for i in range(3):
                _, b = orders[i][STEPS - 1 - j]
                sib = base[i] + sz * (1 - 2 * b)
                new = base[i] - sz * b
                rdma[i].wait()
                if j + 1 < STEPS:
                    accs[i][pl.ds(sib, sz), :] = ag_bufs[i * STEPS + j][...]
                    p, _ = orders[i][STEPS - 2 - j]
                    rdma[i] = mk(
                        accs[i].at[pl.ds(new, 2 * sz), :],
                        ag_bufs[i * STEPS + j + 1],
                        ag_send_sems.at[i * STEPS + j + 1],
                        ag_recv_sems.at[i * STEPS + j + 1], p,
                    )
                    rdma[i].start()
                out_ref[pl.ds(sib, sz), pl.ds(COLS[i], WIDTHS[i])] = (
                    ag_bufs[i * STEPS + j][...].astype(jnp.float32)
                )
                base[i] = new

    rs_shapes = [
        pltpu.VMEM((M >> (s + 1), WIDTHS[i]), jnp.bfloat16)
        for i in range(3) for s in range(STEPS)
    ]
    ag_shapes = [
        pltpu.VMEM((CH << j, WIDTHS[i]), jnp.bfloat16)
        for i in range(3) for j in range(STEPS)
    ]
    return pl.pallas_call(
        body,
        out_shape=jax.ShapeDtypeStruct((M, N), jnp.float32),
        in_specs=[
            pl.BlockSpec(memory_space=pltpu.VMEM),
            pl.BlockSpec(memory_space=pltpu.VMEM),
        ],
        out_specs=pl.BlockSpec(memory_space=pltpu.VMEM),
        scratch_shapes=[
            [pltpu.VMEM((M, WIDTHS[i]), jnp.bfloat16) for i in range(3)],
            pltpu.VMEM((K, N), jnp.bfloat16),
            rs_shapes,
            ag_shapes,
            pltpu.VMEM((CH, WIDTHS[1]), jnp.bfloat16),
            pltpu.VMEM((CH, WIDTHS[2]), jnp.bfloat16),
            pltpu.VMEM((CH, WIDTHS[1]), jnp.bfloat16),
            pltpu.VMEM((CH, WIDTHS[2]), jnp.bfloat16),
            pltpu.VMEM((CH, WIDTHS[1]), jnp.bfloat16),
            pltpu.VMEM((CH, WIDTHS[2]), jnp.bfloat16),
            pltpu.SemaphoreType.DMA((3 * STEPS,)),
            pltpu.SemaphoreType.DMA((3 * STEPS,)),
            pltpu.SemaphoreType.DMA((3 * STEPS,)),
            pltpu.SemaphoreType.DMA((3 * STEPS,)),
            pltpu.SemaphoreType.DMA((4,)),
            pltpu.SemaphoreType.DMA((4,)),
        ],
        compiler_params=pltpu.CompilerParams(collective_id=0),
    )(t, W)


# device time: 81269 ns/iter; 2.6686x vs baseline; 1.0007x over previous
import jax
import jax.numpy as jnp
from jax import lax
from jax.experimental import pallas as pl
from jax.experimental.pallas import tpu as pltpu

M = 2048
K = 1024
N = 1024
CH = 64
WIDTHS = (384, 256, 384)
COLS = (0, 384, 640)
ORD = ("xyz", "yzx", "zxy")
_SHRINK = {"x": 2, "y": 4, "z": 4}
_W = {"x": 1, "y": 3, "z": 3}


def _sizes(order):
    s, out = M, []
    for k in order:
        out.append(s)
        s //= _SHRINK[k]
    return out


def _offs(order):
    o, out = 0, []
    for k in order:
        out.append(o)
        o += _W[k]
    return out


RS_SIZES = [_sizes(o) for o in ORD]
AG_ORD = [o[::-1] for o in ORD]
AG_SIZES = []
for i in range(3):
    s, row = CH, []
    for k in AG_ORD[i]:
        row.append(s)
        s *= _SHRINK[k]
    AG_SIZES.append(row)
RS_OFFS = [_offs(o) for o in ORD]
AG_OFFS = [_offs(o) for o in AG_ORD]


def kernel(t, W):
    def body(t_ref, w_ref, out_ref,
             accs, wb_ref,
             rs_bufs, ag_bufs,
             shufin_b, shufin_c, shufout_b, shufout_c,
             shufin2_b, shufin2_c,
             rs_send_sems, rs_recv_sems,
             ag_send_sems, ag_recv_sems,
             sh_send_sems, sh_recv_sems):
        my = lax.axis_index("i")
        z = my // 8
        r = my - z * 8
        y = r // 2
        q = r - y * 2
        x = jnp.where(y % 2 == 0, q, 1 - q)

        def lid(xx, yy, zz):
            qq = jnp.where(yy % 2 == 0, xx, 1 - xx)
            return zz * 8 + yy * 2 + qq

        def axis_coord(kind):
            return {"x": x, "y": y, "z": z}[kind]

        def axis_partner(kind, ct):
            if kind == "x":
                return lid(1 - x, y, z)
            if kind == "y":
                return lid(x, ct, z)
            return lid(x, y, ct)

        Dp_b = lid(z % 2, 2 * x + y // 2, 2 * (y % 2) + z // 2)
        Dst_b = lid(y // 2, 2 * (y % 2) + z // 2, 2 * (z % 2) + x)
        Dp_c = lid(y % 2, z, 2 * x + y // 2)
        Dst_c = lid(z // 2, 2 * (z % 2) + x, y)
        pred_b = Dp_b != my
        pred_c = Dp_c != my

        barrier_sem = pltpu.get_barrier_semaphore()
        pl.semaphore_signal(
            barrier_sem, inc=1,
            device_id=(lid(1 - x, y, z),),
            device_id_type=pl.DeviceIdType.MESH,
        )
        for k in (1, 2, 3):
            for p in (lid(x, jnp.remainder(y + k, 4), z),
                      lid(x, y, jnp.remainder(z + k, 4))):
                pl.semaphore_signal(
                    barrier_sem, inc=1,
                    device_id=(p,), device_id_type=pl.DeviceIdType.MESH,
                )

        @pl.when(pred_b)
        def _():
            for p in (Dp_b, Dst_b):
                pl.semaphore_signal(
                    barrier_sem, inc=1,
                    device_id=(p,), device_id_type=pl.DeviceIdType.MESH,
                )

        @pl.when(jnp.logical_not(pred_b))
        def _():
            pl.semaphore_signal(barrier_sem, inc=2)

        @pl.when(pred_c)
        def _():
            for p in (Dp_c, Dst_c):
                pl.semaphore_signal(
                    barrier_sem, inc=1,
                    device_id=(p,), device_id_type=pl.DeviceIdType.MESH,
                )

        @pl.when(jnp.logical_not(pred_c))
        def _():
            pl.semaphore_signal(barrier_sem, inc=2)

        pl.semaphore_wait(barrier_sem, 11)

        def mk(src, dst, ssem, rsem, dev):
            return pltpu.make_async_remote_copy(
                src_ref=src, dst_ref=dst, send_sem=ssem, recv_sem=rsem,
                device_id=(dev,), device_id_type=pl.DeviceIdType.MESH,
            )

        def rs_start(i, p, base):
            kind = ORD[i][p]
            S = RS_SIZES[i][p]
            off = i * 7 + RS_OFFS[i][p]
            buf = rs_bufs[i * 3 + p]
            if kind == "x":
                half = S // 2
                d = mk(accs[i].at[pl.ds(base + half * (1 - x), half), :], buf,
                       rs_send_sems.at[off], rs_recv_sems.at[off],
                       axis_partner(kind, None))
                d.start()
                return [d], base + half * x
            c = axis_coord(kind)
            S4 = S // 4
            ds_ = []
            for k in (1, 2, 3):
                ct = jnp.remainder(c + k, 4)
                slot = 3 - k
                d = mk(accs[i].at[pl.ds(base + ct * S4, S4), :], buf.at[slot],
                       rs_send_sems.at[off + k - 1],
                       rs_recv_sems.at[off + slot],
                       axis_partner(kind, ct))
                d.start()
                ds_.append(d)
            return ds_, base + c * S4

        def rs_finish(i, p, ds_, kept):
            kind = ORD[i][p]
            S = RS_SIZES[i][p]
            for d in ds_:
                d.wait()
            buf = rs_bufs[i * 3 + p]
            if kind == "x":
                half = S // 2
                accs[i][pl.ds(kept, half), :] = (
                    accs[i][pl.ds(kept, half), :] + buf[...]
                )
            else:
                S4 = S // 4
                accs[i][pl.ds(kept, S4), :] = (
                    accs[i][pl.ds(kept, S4), :]
                    + buf[0] + buf[1] + buf[2]
                )

        def ag_start(i, j, base):
            kind = AG_ORD[i][j]
            sz = AG_SIZES[i][j]
            off = i * 7 + AG_OFFS[i][j]
            buf = ag_bufs[i * 3 + j]
            if kind == "x":
                d = mk(accs[i].at[pl.ds(base, sz), :], buf,
                       ag_send_sems.at[off], ag_recv_sems.at[off],
                       axis_partner(kind, None))
                d.start()
                return [d], base - sz * x
            c = axis_coord(kind)
            ds_ = []
            for k in (1, 2, 3):
                ct = jnp.remainder(c + k, 4)
                slot = 3 - k
                d = mk(accs[i].at[pl.ds(base, sz), :], buf.at[slot],
                       ag_send_sems.at[off + k - 1],
                       ag_recv_sems.at[off + slot],
                       axis_partner(kind, ct))
                d.start()
                ds_.append(d)
            return ds_, base - c * sz

        def ag_finish(i, j, ds_, base_old, merged, last):
            kind = AG_ORD[i][j]
            sz = AG_SIZES[i][j]
            buf = ag_bufs[i * 3 + j]
            ci, wi = COLS[i], WIDTHS[i]
            for d in ds_:
                d.wait()
            if kind == "x":
                sib = base_old + sz * (1 - 2 * x)
                if not last:
                    accs[i][pl.ds(sib, sz), :] = buf[...]
                out_ref[pl.ds(sib, sz), pl.ds(ci, wi)] = buf[...].astype(
                    jnp.float32
                )
            else:
                c = axis_coord(kind)
                for s in range(3):
                    cs = jnp.remainder(c + s + 1, 4)
                    pos = merged + cs * sz
                    if not last:
                        accs[i][pl.ds(pos, sz), :] = buf[s]
                    out_ref[pl.ds(pos, sz), pl.ds(ci, wi)] = buf[s].astype(
                        jnp.float32
                    )

        rdma = [None, None, None]
        base = [None, None, None]
        kept0 = [None, None, None]
        for i in range(3):
            kind = ORD[i][0]
            ci, wi = COLS[i], WIDTHS[i]
            if kind == "x":
                half = M // 2
                sb = half * (1 - x)
                accs[i][pl.ds(sb, half), :] = t_ref[
                    pl.ds(sb, half), ci:ci + wi
                ].astype(jnp.bfloat16)
                kept0[i] = jnp.int32(half) * x
            else:
                c = axis_coord(kind)
                S4 = M // 4
                for k in (1, 2, 3):
                    ct = jnp.remainder(c + k, 4)
                    accs[i][pl.ds(ct * S4, S4), :] = t_ref[
                        pl.ds(ct * S4, S4), ci:ci + wi
                    ].astype(jnp.bfloat16)
                kept0[i] = c * S4
            rdma[i], base[i] = rs_start(i, 0, jnp.int32(0))
        for i in range(3):
            ci, wi = COLS[i], WIDTHS[i]
            keep_sz = M // _SHRINK[ORD[i][0]]
            accs[i][pl.ds(kept0[i], keep_sz), :] = t_ref[
                pl.ds(kept0[i], keep_sz), ci:ci + wi
            ].astype(jnp.bfloat16)
        wb_ref[...] = w_ref[...].astype(jnp.bfloat16)

        for p in range(3):
            for i in range(3):
                rs_finish(i, p, rdma[i], base[i])
                if p + 1 < 3:
                    rdma[i], base[i] = rs_start(i, p + 1, base[i])

        baseA, baseB, baseC = base

        rdma_pre_b = mk(accs[1].at[pl.ds(baseB, CH), :], shufin_b,
                        sh_send_sems.at[0], sh_recv_sems.at[0], Dst_b)
        rdma_pre_c = mk(accs[2].at[pl.ds(baseC, CH), :], shufin_c,
                        sh_send_sems.at[1], sh_recv_sems.at[1], Dst_c)

        @pl.when(pred_b)
        def _():
            rdma_pre_b.start()

        @pl.when(jnp.logical_not(pred_b))
        def _():
            shufin_b[...] = accs[1][pl.ds(baseB, CH), :]

        @pl.when(pred_c)
        def _():
            rdma_pre_c.start()

        @pl.when(jnp.logical_not(pred_c))
        def _():
            shufin_c[...] = accs[2][pl.ds(baseC, CH), :]

        yv = jnp.dot(
            accs[0][pl.ds(baseA, CH), :], wb_ref[0:WIDTHS[0], :],
            preferred_element_type=jnp.float32,
        )

        @pl.when(pred_b)
        def _():
            rdma_pre_b.wait()

        yv = yv + jnp.dot(
            shufin_b[...], wb_ref[COLS[1]:COLS[1] + WIDTHS[1], :],
            preferred_element_type=jnp.float32,
        )

        @pl.when(pred_c)
        def _():
            rdma_pre_c.wait()

        yv = yv + jnp.dot(
            shufin_c[...], wb_ref[COLS[2]:K, :],
            preferred_element_type=jnp.float32,
        )
        accs[0][pl.ds(baseA, CH), :] = yv[:, 0:WIDTHS[0]].astype(jnp.bfloat16)
        shufout_b[...] = yv[:, COLS[1]:COLS[1] + WIDTHS[1]].astype(
            jnp.bfloat16
        )
        shufout_c[...] = yv[:, COLS[2]:N].astype(jnp.bfloat16)

        rdma_post_b = mk(shufout_b, shufin2_b,
                         sh_send_sems.at[2], sh_recv_sems.at[2], Dp_b)
        rdma_post_c = mk(shufout_c, shufin2_c,
                         sh_send_sems.at[3], sh_recv_sems.at[3], Dp_c)

        @pl.when(pred_b)
        def _():
            rdma_post_b.start()

        @pl.when(jnp.logical_not(pred_b))
        def _():
            shufin2_b[...] = shufout_b[...]

        @pl.when(pred_c)
        def _():
            rdma_post_c.start()

        @pl.when(jnp.logical_not(pred_c))
        def _():
            shufin2_c[...] = shufout_c[...]

        merged = [None, None, None]
        rdma[0], merged[0] = ag_start(0, 0, baseA)
        out_ref[pl.ds(baseA, CH), :] = yv

        @pl.when(pred_b)
        def _():
            rdma_post_b.wait()

        accs[1][pl.ds(baseB, CH), :] = shufin2_b[...]
        rdma[1], merged[1] = ag_start(1, 0, baseB)
        out_ref[pl.ds(baseB, CH), pl.ds(COLS[1], WIDTHS[1])] = (
            shufin2_b[...].astype(jnp.float32)
        )

        @pl.when(pred_c)
        def _():
            rdma_post_c.wait()

        accs[2][pl.ds(baseC, CH), :] = shufin2_c[...]
        rdma[2], merged[2] = ag_start(2, 0, baseC)
        out_ref[pl.ds(baseC, CH), pl.ds(COLS[2], WIDTHS[2])] = (
            shufin2_c[...].astype(jnp.float32)
        )

        base = [baseA, baseB, baseC]

        for j in range(3):
            for i in range(3):
                ag_finish(i, j, rdma[i], base[i], merged[i], last=(j == 2))
                base[i] = merged[i]
                if j + 1 < 3:
                    rdma[i], merged[i] = ag_start(i, j + 1, base[i])

    def _rs_buf(i, p):
        kind = ORD[i][p]
        S = RS_SIZES[i][p]
        if kind == "x":
            return pltpu.VMEM((S // 2, WIDTHS[i]), jnp.bfloat16)
        return pltpu.VMEM((3, S // 4, WIDTHS[i]), jnp.bfloat16)

    def _ag_buf(i, j):
        kind = AG_ORD[i][j]
        sz = AG_SIZES[i][j]
        if kind == "x":
            return pltpu.VMEM((sz, WIDTHS[i]), jnp.bfloat16)
        return pltpu.VMEM((3, sz, WIDTHS[i]), jnp.bfloat16)

    rs_shapes = [_rs_buf(i, p) for i in range(3) for p in range(3)]
    ag_shapes = [_ag_buf(i, j) for i in range(3) for j in range(3)]
    return pl.pallas_call(
        body,
        out_shape=jax.ShapeDtypeStruct((M, N), jnp.float32),
        in_specs=[
            pl.BlockSpec(memory_space=pltpu.VMEM),
            pl.BlockSpec(memory_space=pltpu.VMEM),
        ],
        out_specs=pl.BlockSpec(memory_space=pltpu.VMEM),
        scratch_shapes=[
            [pltpu.VMEM((M, WIDTHS[i]), jnp.bfloat16) for i in range(3)],
            pltpu.VMEM((K, N), jnp.bfloat16),
            rs_shapes,
            ag_shapes,
            pltpu.VMEM((CH, WIDTHS[1]), jnp.bfloat16),
            pltpu.VMEM((CH, WIDTHS[2]), jnp.bfloat16),
            pltpu.VMEM((CH, WIDTHS[1]), jnp.bfloat16),
            pltpu.VMEM((CH, WIDTHS[2]), jnp.bfloat16),
            pltpu.VMEM((CH, WIDTHS[1]), jnp.bfloat16),
            pltpu.VMEM((CH, WIDTHS[2]), jnp.bfloat16),
            pltpu.SemaphoreType.DMA((21,)),
            pltpu.SemaphoreType.DMA((21,)),
            pltpu.SemaphoreType.DMA((21,)),
            pltpu.SemaphoreType.DMA((21,)),
            pltpu.SemaphoreType.DMA((4,)),
            pltpu.SemaphoreType.DMA((4,)),
        ],
        compiler_params=pltpu.CompilerParams(collective_id=0),
    )(t, W)
